# initial kernel scaffold (unmeasured)
import jax
import jax.numpy as jnp
from jax import lax
from jax.experimental import pallas as pl
from jax.experimental.pallas import tpu as pltpu

B, S, D = 2, 512, 2048
H, Dh, Dr = 16, 128, 32
BS = B * S
SCALE = (Dh + Dr) ** -0.5
F32 = jnp.float32


def _kv_exchange_body(x_ref, wdkv_ref, wuk_ref, wuv_ref, k_ref, v_ref,
                      rbuf, send_sems, recv_sems):
    my_x = lax.axis_index("x")
    my_y = lax.axis_index("y")
    nbr = (1 - my_x, my_y)

    barrier = pltpu.get_barrier_semaphore()
    pl.semaphore_signal(barrier, inc=1, device_id=nbr,
                        device_id_type=pl.DeviceIdType.MESH)
    pl.semaphore_wait(barrier, 1)

    c = jnp.dot(x_ref[...], wdkv_ref[...], preferred_element_type=F32)
    k_ref[...] = jnp.dot(c, wuk_ref[...], preferred_element_type=F32)
    v_ref[...] = jnp.dot(c, wuv_ref[...], preferred_element_type=F32)

    rdma_k = pltpu.make_async_remote_copy(
        src_ref=k_ref, dst_ref=rbuf.at[0],
        send_sem=send_sems.at[0], recv_sem=recv_sems.at[0],
        device_id=nbr, device_id_type=pl.DeviceIdType.MESH)
    rdma_v = pltpu.make_async_remote_copy(
        src_ref=v_ref, dst_ref=rbuf.at[1],
        send_sem=send_sems.at[1], recv_sem=recv_sems.at[1],
        device_id=nbr, device_id_type=pl.DeviceIdType.MESH)
    rdma_k.start()
    rdma_v.start()
    rdma_k.wait()
    rdma_v.wait()

    k_ref[...] = k_ref[...] + rbuf[0, :, :]
    v_ref[...] = v_ref[...] + rbuf[1, :, :]


def _kv_exchange(x2d, Wdkv, Wuk, Wuv):
    return pl.pallas_call(
        _kv_exchange_body,
        out_shape=[jax.ShapeDtypeStruct((BS, D), F32)] * 2,
        in_specs=[pl.BlockSpec(memory_space=pltpu.VMEM)] * 4,
        out_specs=[pl.BlockSpec(memory_space=pltpu.VMEM)] * 2,
        scratch_shapes=[
            pltpu.VMEM((2, BS, D), F32),
            pltpu.SemaphoreType.DMA((2,)),
            pltpu.SemaphoreType.DMA((2,)),
        ],
        compiler_params=pltpu.CompilerParams(collective_id=0),
    )(x2d, Wdkv, Wuk, Wuv)


def _proj_body(x_ref, wq_ref, wqr_ref, wkr_ref, q_ref, qr_ref, kr_ref):
    x = x_ref[...]
    q_ref[...] = jnp.dot(x, wq_ref[...], preferred_element_type=F32)
    qr_ref[...] = jnp.dot(x, wqr_ref[...], preferred_element_type=F32)
    kr_ref[...] = jnp.dot(x, wkr_ref[...], preferred_element_type=F32)


def _proj(x2d, Wq, Wqr, Wkr):
    return pl.pallas_call(
        _proj_body,
        out_shape=[
            jax.ShapeDtypeStruct((BS, H * Dh), F32),
            jax.ShapeDtypeStruct((BS, H * Dr), F32),
            jax.ShapeDtypeStruct((BS, Dr), F32),
        ],
        in_specs=[pl.BlockSpec(memory_space=pltpu.VMEM)] * 4,
        out_specs=[pl.BlockSpec(memory_space=pltpu.VMEM)] * 3,
    )(x2d, Wq, Wqr, Wkr)


def _attn_body(q_ref, k_ref, v_ref, qr_ref, kr_ref, o_ref):
    dims = (((1,), (1,)), ((), ()))
    s = (lax.dot_general(q_ref[...], k_ref[...], dims,
                         preferred_element_type=F32)
         + lax.dot_general(qr_ref[...], kr_ref[...], dims,
                           preferred_element_type=F32)) * SCALE
    m = jnp.max(s, axis=-1, keepdims=True)
    p = jnp.exp(s - m)
    p = p / jnp.sum(p, axis=-1, keepdims=True)
    o_ref[...] = jnp.dot(p, v_ref[...], preferred_element_type=F32)


def _attn(q2d, k2d, v2d, qr2d, kr2d):
    return pl.pallas_call(
        _attn_body,
        grid=(B, H),
        out_shape=jax.ShapeDtypeStruct((BS, H * Dh), F32),
        in_specs=[
            pl.BlockSpec((S, Dh), lambda b, h: (b, h)),
            pl.BlockSpec((S, Dh), lambda b, h: (b, h)),
            pl.BlockSpec((S, Dh), lambda b, h: (b, h)),
            pl.BlockSpec((S, Dr), lambda b, h: (b, h)),
            pl.BlockSpec((S, Dr), lambda b, h: (b, 0)),
        ],
        out_specs=pl.BlockSpec((S, Dh), lambda b, h: (b, h)),
    )(q2d, k2d, v2d, qr2d, kr2d)


def _out_body(o_ref, wo_ref, out_ref):
    out_ref[...] = jnp.dot(o_ref[...], wo_ref[...], preferred_element_type=F32)


def _outproj(o2d, Wo):
    return pl.pallas_call(
        _out_body,
        out_shape=jax.ShapeDtypeStruct((BS, D), F32),
        in_specs=[pl.BlockSpec(memory_space=pltpu.VMEM)] * 2,
        out_specs=pl.BlockSpec(memory_space=pltpu.VMEM),
    )(o2d, Wo)


def kernel(x, Wdkv, Wuk, Wuv, Wq, Wqr, Wkr, Wo):
    x2d = x.reshape(BS, D)
    k2d, v2d = _kv_exchange(x2d, Wdkv, Wuk, Wuv)
    q2d, qr2d, kr2d = _proj(x2d, Wq, Wqr, Wkr)
    o2d = _attn(q2d, k2d, v2d, qr2d, kr2d)
    out2d = _outproj(o2d, Wo)
    return out2d.reshape(B, S, D)


# baseline (device time: 288533 ns/iter reference)
import jax
import jax.numpy as jnp
from jax import lax
from jax.experimental import pallas as pl
from jax.experimental.pallas import tpu as pltpu

B, S, D = 2, 512, 2048
H, Dh, Dr = 16, 128, 32
BS = B * S
SCALE = (Dh + Dr) ** -0.5
F32 = jnp.float32


def _kv_exchange_body(x_ref, wdkv_ref, wuk_ref, wuv_ref, k_ref, v_ref,
                      rbuf, send_sems, recv_sems):
    my_x = lax.axis_index("x")
    my_y = lax.axis_index("y")
    nbr = (1 - my_x, my_y)

    barrier = pltpu.get_barrier_semaphore()
    pl.semaphore_signal(barrier, inc=1, device_id=nbr,
                        device_id_type=pl.DeviceIdType.MESH)
    pl.semaphore_wait(barrier, 1)

    c = jnp.dot(x_ref[...], wdkv_ref[...], preferred_element_type=F32)
    k_ref[...] = jnp.dot(c, wuk_ref[...], preferred_element_type=F32)
    v_ref[...] = jnp.dot(c, wuv_ref[...], preferred_element_type=F32)

    rdma_k = pltpu.make_async_remote_copy(
        src_ref=k_ref, dst_ref=rbuf.at[0],
        send_sem=send_sems.at[0], recv_sem=recv_sems.at[0],
        device_id=nbr, device_id_type=pl.DeviceIdType.MESH)
    rdma_v = pltpu.make_async_remote_copy(
        src_ref=v_ref, dst_ref=rbuf.at[1],
        send_sem=send_sems.at[1], recv_sem=recv_sems.at[1],
        device_id=nbr, device_id_type=pl.DeviceIdType.MESH)
    rdma_k.start()
    rdma_v.start()
    rdma_k.wait()
    rdma_v.wait()

    k_ref[...] = k_ref[...] + rbuf[0, :, :]
    v_ref[...] = v_ref[...] + rbuf[1, :, :]


def _kv_exchange(x2d, Wdkv, Wuk, Wuv):
    return pl.pallas_call(
        _kv_exchange_body,
        out_shape=[jax.ShapeDtypeStruct((BS, D), F32)] * 2,
        in_specs=[pl.BlockSpec(memory_space=pltpu.VMEM)] * 4,
        out_specs=[pl.BlockSpec(memory_space=pltpu.VMEM)] * 2,
        scratch_shapes=[
            pltpu.VMEM((2, BS, D), F32),
            pltpu.SemaphoreType.DMA((2,)),
            pltpu.SemaphoreType.DMA((2,)),
        ],
        compiler_params=pltpu.CompilerParams(collective_id=0),
    )(x2d, Wdkv, Wuk, Wuv)


def _proj_body(x_ref, wq_ref, wqr_ref, wkr_ref, q_ref, qr_ref, kr_ref):
    x = x_ref[...]
    q_ref[...] = jnp.dot(x, wq_ref[...], preferred_element_type=F32)
    qr_ref[...] = jnp.dot(x, wqr_ref[...], preferred_element_type=F32)
    kr_ref[...] = jnp.dot(x, wkr_ref[...], preferred_element_type=F32)


def _proj(x2d, Wq, Wqr, Wkr):
    return pl.pallas_call(
        _proj_body,
        out_shape=[
            jax.ShapeDtypeStruct((BS, H * Dh), F32),
            jax.ShapeDtypeStruct((BS, H * Dr), F32),
            jax.ShapeDtypeStruct((BS, Dr), F32),
        ],
        in_specs=[pl.BlockSpec(memory_space=pltpu.VMEM)] * 4,
        out_specs=[pl.BlockSpec(memory_space=pltpu.VMEM)] * 3,
    )(x2d, Wq, Wqr, Wkr)


def _attn_body(q_ref, k_ref, v_ref, qr_ref, kr_ref, o_ref):
    dims = (((1,), (1,)), ((), ()))
    s = (lax.dot_general(q_ref[...], k_ref[...], dims,
                         preferred_element_type=F32)
         + lax.dot_general(qr_ref[0], kr_ref[...], dims,
                           preferred_element_type=F32)) * SCALE
    m = jnp.max(s, axis=-1, keepdims=True)
    p = jnp.exp(s - m)
    p = p / jnp.sum(p, axis=-1, keepdims=True)
    o_ref[...] = jnp.dot(p, v_ref[...], preferred_element_type=F32)


def _attn(q2d, k2d, v2d, qr3d, kr2d):
    return pl.pallas_call(
        _attn_body,
        grid=(B, H),
        out_shape=jax.ShapeDtypeStruct((BS, H * Dh), F32),
        in_specs=[
            pl.BlockSpec((S, Dh), lambda b, h: (b, h)),
            pl.BlockSpec((S, Dh), lambda b, h: (b, h)),
            pl.BlockSpec((S, Dh), lambda b, h: (b, h)),
            pl.BlockSpec((1, S, Dr), lambda b, h: (h, b, 0)),
            pl.BlockSpec((S, Dr), lambda b, h: (b, 0)),
        ],
        out_specs=pl.BlockSpec((S, Dh), lambda b, h: (b, h)),
    )(q2d, k2d, v2d, qr3d, kr2d)


def _out_body(o_ref, wo_ref, out_ref):
    out_ref[...] = jnp.dot(o_ref[...], wo_ref[...], preferred_element_type=F32)


def _outproj(o2d, Wo):
    return pl.pallas_call(
        _out_body,
        out_shape=jax.ShapeDtypeStruct((BS, D), F32),
        in_specs=[pl.BlockSpec(memory_space=pltpu.VMEM)] * 2,
        out_specs=pl.BlockSpec(memory_space=pltpu.VMEM),
    )(o2d, Wo)


def kernel(x, Wdkv, Wuk, Wuv, Wq, Wqr, Wkr, Wo):
    x2d = x.reshape(BS, D)
    k2d, v2d = _kv_exchange(x2d, Wdkv, Wuk, Wuv)
    q2d, qr2d, kr2d = _proj(x2d, Wq, Wqr, Wkr)
    qr3d = qr2d.reshape(BS, H, Dr).transpose(1, 0, 2)
    o2d = _attn(q2d, k2d, v2d, qr3d, kr2d)
    out2d = _outproj(o2d, Wo)
    return out2d.reshape(B, S, D)


# device time: 168936 ns/iter; 1.7079x vs baseline; 1.7079x over previous
import jax
import jax.numpy as jnp
from jax import lax
from jax.experimental import pallas as pl
from jax.experimental.pallas import tpu as pltpu

B, S, D = 2, 512, 2048
H, Dh, Dr = 16, 128, 32
BS = B * S
HG = H // 4
CW = HG * Dh
RW = HG * Dr
SCALE = (Dh + Dr) ** -0.5
F32 = jnp.float32

_MESH = pl.DeviceIdType.MESH
_VMEM_LIMIT = 56 * 1024 * 1024


def _ring_pos(x, y):
    return 2 * x + (x ^ y)


def _ring_coords(q):
    return (q // 2, (q ^ (q // 2)) & 1)


def _attn_body(x_ref, wdkv_ref, wuk_ref, wuv_ref, wq_ref, wqr_ref, wkr_ref,
               o_ref,
               kvown, kvsend, kvrecv, q_scr, qr_scr, kr_scr,
               kv_send_sem, kv_recv_sem):
    my_x = lax.axis_index("x")
    my_y = lax.axis_index("y")
    p = _ring_pos(my_x, my_y)
    pp = _ring_pos(1 - my_x, my_y)

    c = jnp.dot(x_ref[...], wdkv_ref[...], preferred_element_type=F32)
    kvown[0] = jnp.dot(c, wuk_ref[:, pl.ds(p * CW, CW)],
                       preferred_element_type=F32)
    kvown[1] = jnp.dot(c, wuv_ref[:, pl.ds(p * CW, CW)],
                       preferred_element_type=F32)
    kvsend[0] = jnp.dot(c, wuk_ref[:, pl.ds(pp * CW, CW)],
                        preferred_element_type=F32)
    kvsend[1] = jnp.dot(c, wuv_ref[:, pl.ds(pp * CW, CW)],
                        preferred_element_type=F32)

    barrier = pltpu.get_barrier_semaphore()
    pl.semaphore_signal(barrier, inc=1, device_id=(1 - my_x, my_y),
                        device_id_type=_MESH)
    pl.semaphore_wait(barrier, 1)

    kv_rdma = pltpu.make_async_remote_copy(
        src_ref=kvsend, dst_ref=kvrecv,
        send_sem=kv_send_sem, recv_sem=kv_recv_sem,
        device_id=(1 - my_x, my_y), device_id_type=_MESH)
    kv_rdma.start()

    x = x_ref[...]
    q_scr[...] = jnp.dot(x, wq_ref[...], preferred_element_type=F32)
    qr_scr[...] = jnp.dot(x, wqr_ref[...], preferred_element_type=F32)
    kr_scr[...] = jnp.dot(x, wkr_ref[...], preferred_element_type=F32)

    kv_rdma.wait()
    kvown[0] = kvown[0] + kvrecv[0]
    kvown[1] = kvown[1] + kvrecv[1]

    dims = (((1,), (1,)), ((), ()))
    for b in range(B):
        rows = slice(b * S, (b + 1) * S)
        kr_b = kr_scr[rows, :]
        qr_b = qr_scr[rows, :]
        for j in range(HG):
            cols = slice(j * Dh, (j + 1) * Dh)
            q_bh = q_scr[rows, cols]
            k_bh = kvown[0, rows, cols]
            qr_bh = qr_b[:, j * Dr:(j + 1) * Dr]
            s = (lax.dot_general(q_bh, k_bh, dims, preferred_element_type=F32)
                 + lax.dot_general(qr_bh, kr_b, dims,
                                   preferred_element_type=F32)) * SCALE
            m = jnp.max(s, axis=-1, keepdims=True)
            pr = jnp.exp(s - m)
            pr = pr / jnp.sum(pr, axis=-1, keepdims=True)
            o_ref[rows, cols] = jnp.dot(pr, kvown[1, rows, cols],
                                        preferred_element_type=F32)


def _ring_body(o_ref, wo_ref, out_ref, obuf, send_sems, recv_sems):
    my_x = lax.axis_index("x")
    my_y = lax.axis_index("y")
    p = _ring_pos(my_x, my_y)
    right = _ring_coords((p + 1) % 4)
    left = _ring_coords((p + 3) % 4)

    barrier = pltpu.get_barrier_semaphore()
    for nbr in (right, left):
        pl.semaphore_signal(barrier, inc=1, device_id=nbr,
                            device_id_type=_MESH)
    pl.semaphore_wait(barrier, 2)

    for h in range(3):
        rdma = pltpu.make_async_remote_copy(
            src_ref=(o_ref if h == 0 else obuf.at[h - 1]),
            dst_ref=obuf.at[h],
            send_sem=send_sems.at[h], recv_sem=recv_sems.at[h],
            device_id=right, device_id_type=_MESH)
        rdma.start()
        origin = (p + 4 - h) % 4
        chunk = o_ref[...] if h == 0 else obuf[h - 1]
        partial = jnp.dot(chunk, wo_ref[pl.ds(origin * CW, CW), :],
                          preferred_element_type=F32)
        if h == 0:
            out_ref[...] = partial
        else:
            out_ref[...] = out_ref[...] + partial
        rdma.wait()

    origin = (p + 1) % 4
    out_ref[...] = out_ref[...] + jnp.dot(
        obuf[2], wo_ref[pl.ds(origin * CW, CW), :],
        preferred_element_type=F32)


def kernel(x, Wdkv, Wuk, Wuv, Wq, Wqr, Wkr, Wo):
    x2d = x.reshape(BS, D)
    my_x = lax.axis_index("x")
    my_y = lax.axis_index("y")
    p = _ring_pos(my_x, my_y)
    wq_s = lax.dynamic_slice(Wq, (0, p * CW), (D, CW))
    wqr_s = lax.dynamic_slice(Wqr, (0, p * RW), (D, RW))

    o_own = pl.pallas_call(
        _attn_body,
        out_shape=jax.ShapeDtypeStruct((BS, CW), F32),
        in_specs=[pl.BlockSpec(memory_space=pltpu.VMEM)] * 7,
        out_specs=pl.BlockSpec(memory_space=pltpu.VMEM),
        scratch_shapes=[
            pltpu.VMEM((2, BS, CW), F32),
            pltpu.VMEM((2, BS, CW), F32),
            pltpu.VMEM((2, BS, CW), F32),
            pltpu.VMEM((BS, CW), F32),
            pltpu.VMEM((BS, RW), F32),
            pltpu.VMEM((BS, Dr), F32),
            pltpu.SemaphoreType.DMA,
            pltpu.SemaphoreType.DMA,
        ],
        compiler_params=pltpu.CompilerParams(
            collective_id=0, vmem_limit_bytes=_VMEM_LIMIT),
    )(x2d, Wdkv, Wuk, Wuv, wq_s, wqr_s, Wkr)

    out2d = pl.pallas_call(
        _ring_body,
        out_shape=jax.ShapeDtypeStruct((BS, D), F32),
        in_specs=[pl.BlockSpec(memory_space=pltpu.VMEM)] * 2,
        out_specs=pl.BlockSpec(memory_space=pltpu.VMEM),
        scratch_shapes=[
            pltpu.VMEM((3, BS, CW), F32),
            pltpu.SemaphoreType.DMA((3,)),
            pltpu.SemaphoreType.DMA((3,)),
        ],
        compiler_params=pltpu.CompilerParams(
            collective_id=1, vmem_limit_bytes=_VMEM_LIMIT),
    )(o_own, Wo)
    return out2d.reshape(B, S, D)


# device time: 135199 ns/iter; 2.1341x vs baseline; 1.2495x over previous
import jax
import jax.numpy as jnp
from jax import lax
from jax.experimental import pallas as pl
from jax.experimental.pallas import tpu as pltpu

B, S, D = 2, 512, 2048
H, Dh, Dr = 16, 128, 32
DC = 128
BS = B * S
HG = H // 4
CW = HG * Dh
RW = HG * Dr
SCALE = (Dh + Dr) ** -0.5
F32 = jnp.float32

_MESH = pl.DeviceIdType.MESH
_VMEM_LIMIT = 56 * 1024 * 1024


def _ring_pos(x, y):
    return 2 * x + (x ^ y)


def _ring_coords(q):
    return (q // 2, (q ^ (q // 2)) & 1)


def _attn_body(x_ref, wdkv_ref, wuk_ref, wuv_ref, wq_ref, wqr_ref, wkr_ref,
               o_ref,
               kvown, csend, crecv, wsend, wrecv, q_scr, qr_scr, kr_scr,
               c_send_sem, c_recv_sem, w_send_sem, w_recv_sem):
    my_x = lax.axis_index("x")
    my_y = lax.axis_index("y")
    p = _ring_pos(my_x, my_y)
    pp = _ring_pos(1 - my_x, my_y)

    csend[...] = jnp.dot(x_ref[...], wdkv_ref[...], preferred_element_type=F32)
    wsend[0] = wuk_ref[:, pl.ds(pp * CW, CW)]
    wsend[1] = wuv_ref[:, pl.ds(pp * CW, CW)]

    barrier = pltpu.get_barrier_semaphore()
    pl.semaphore_signal(barrier, inc=1, device_id=(1 - my_x, my_y),
                        device_id_type=_MESH)
    pl.semaphore_wait(barrier, 1)

    c_rdma = pltpu.make_async_remote_copy(
        src_ref=csend, dst_ref=crecv,
        send_sem=c_send_sem, recv_sem=c_recv_sem,
        device_id=(1 - my_x, my_y), device_id_type=_MESH)
    w_rdma = pltpu.make_async_remote_copy(
        src_ref=wsend, dst_ref=wrecv,
        send_sem=w_send_sem, recv_sem=w_recv_sem,
        device_id=(1 - my_x, my_y), device_id_type=_MESH)
    c_rdma.start()
    w_rdma.start()

    x = x_ref[...]
    q_scr[...] = jnp.dot(x, wq_ref[...], preferred_element_type=F32)
    qr_scr[...] = jnp.dot(x, wqr_ref[...], preferred_element_type=F32)
    kr_scr[...] = jnp.dot(x, wkr_ref[...], preferred_element_type=F32)

    c_rdma.wait()
    w_rdma.wait()
    kvown[0] = (jnp.dot(csend[...], wuk_ref[:, pl.ds(p * CW, CW)],
                        preferred_element_type=F32)
                + jnp.dot(crecv[...], wrecv[0],
                          preferred_element_type=F32))
    kvown[1] = (jnp.dot(csend[...], wuv_ref[:, pl.ds(p * CW, CW)],
                        preferred_element_type=F32)
                + jnp.dot(crecv[...], wrecv[1],
                          preferred_element_type=F32))

    dims = (((1,), (1,)), ((), ()))
    for b in range(B):
        rows = slice(b * S, (b + 1) * S)
        kr_b = kr_scr[rows, :]
        qr_b = qr_scr[rows, :]
        for j in range(HG):
            cols = slice(j * Dh, (j + 1) * Dh)
            q_bh = q_scr[rows, cols]
            k_bh = kvown[0, rows, cols]
            qr_bh = qr_b[:, j * Dr:(j + 1) * Dr]
            s = (lax.dot_general(q_bh, k_bh, dims, preferred_element_type=F32)
                 + lax.dot_general(qr_bh, kr_b, dims,
                                   preferred_element_type=F32)) * SCALE
            m = jnp.max(s, axis=-1, keepdims=True)
            pr = jnp.exp(s - m)
            pr = pr / jnp.sum(pr, axis=-1, keepdims=True)
            o_ref[rows, cols] = jnp.dot(pr, kvown[1, rows, cols],
                                        preferred_element_type=F32)


def _ring_body(o_ref, wo_ref, out_ref, obuf, send_sems, recv_sems):
    my_x = lax.axis_index("x")
    my_y = lax.axis_index("y")
    p = _ring_pos(my_x, my_y)
    right = _ring_coords((p + 1) % 4)
    left = _ring_coords((p + 3) % 4)

    barrier = pltpu.get_barrier_semaphore()
    for nbr in (right, left):
        pl.semaphore_signal(barrier, inc=1, device_id=nbr,
                            device_id_type=_MESH)
    pl.semaphore_wait(barrier, 2)

    for h in range(3):
        rdma = pltpu.make_async_remote_copy(
            src_ref=(o_ref if h == 0 else obuf.at[h - 1]),
            dst_ref=obuf.at[h],
            send_sem=send_sems.at[h], recv_sem=recv_sems.at[h],
            device_id=right, device_id_type=_MESH)
        rdma.start()
        origin = (p + 4 - h) % 4
        chunk = o_ref[...] if h == 0 else obuf[h - 1]
        partial = jnp.dot(chunk, wo_ref[pl.ds(origin * CW, CW), :],
                          preferred_element_type=F32)
        if h == 0:
            out_ref[...] = partial
        else:
            out_ref[...] = out_ref[...] + partial
        rdma.wait()

    origin = (p + 1) % 4
    out_ref[...] = out_ref[...] + jnp.dot(
        obuf[2], wo_ref[pl.ds(origin * CW, CW), :],
        preferred_element_type=F32)


def kernel(x, Wdkv, Wuk, Wuv, Wq, Wqr, Wkr, Wo):
    x2d = x.reshape(BS, D)
    my_x = lax.axis_index("x")
    my_y = lax.axis_index("y")
    p = _ring_pos(my_x, my_y)
    wq_s = lax.dynamic_slice(Wq, (0, p * CW), (D, CW))
    wqr_s = lax.dynamic_slice(Wqr, (0, p * RW), (D, RW))

    o_own = pl.pallas_call(
        _attn_body,
        out_shape=jax.ShapeDtypeStruct((BS, CW), F32),
        in_specs=[pl.BlockSpec(memory_space=pltpu.VMEM)] * 7,
        out_specs=pl.BlockSpec(memory_space=pltpu.VMEM),
        scratch_shapes=[
            pltpu.VMEM((2, BS, CW), F32),
            pltpu.VMEM((BS, DC), F32),
            pltpu.VMEM((BS, DC), F32),
            pltpu.VMEM((2, DC, CW), F32),
            pltpu.VMEM((2, DC, CW), F32),
            pltpu.VMEM((BS, CW), F32),
            pltpu.VMEM((BS, RW), F32),
            pltpu.VMEM((BS, Dr), F32),
            pltpu.SemaphoreType.DMA,
            pltpu.SemaphoreType.DMA,
            pltpu.SemaphoreType.DMA,
            pltpu.SemaphoreType.DMA,
        ],
        compiler_params=pltpu.CompilerParams(
            collective_id=0, vmem_limit_bytes=_VMEM_LIMIT),
    )(x2d, Wdkv, Wuk, Wuv, wq_s, wqr_s, Wkr)

    out2d = pl.pallas_call(
        _ring_body,
        out_shape=jax.ShapeDtypeStruct((BS, D), F32),
        in_specs=[pl.BlockSpec(memory_space=pltpu.VMEM)] * 2,
        out_specs=pl.BlockSpec(memory_space=pltpu.VMEM),
        scratch_shapes=[
            pltpu.VMEM((3, BS, CW), F32),
            pltpu.SemaphoreType.DMA((3,)),
            pltpu.SemaphoreType.DMA((3,)),
        ],
        compiler_params=pltpu.CompilerParams(
            collective_id=1, vmem_limit_bytes=_VMEM_LIMIT),
    )(o_own, Wo)
    return out2d.reshape(B, S, D)


# device time: 121781 ns/iter; 2.3693x vs baseline; 1.1102x over previous
import jax
import jax.numpy as jnp
from jax import lax
from jax.experimental import pallas as pl
from jax.experimental.pallas import tpu as pltpu

B, S, D = 2, 512, 2048
H, Dh, Dr = 16, 128, 32
DC = 128
BS = B * S
HG = H // 4
CW = HG * Dh
RW = HG * Dr
SCALE = (Dh + Dr) ** -0.5
F32 = jnp.float32

_MESH = pl.DeviceIdType.MESH


def _ring_pos(x, y):
    return 2 * x + (x ^ y)


def _ring_coords(q):
    return (q // 2, (q ^ (q // 2)) & 1)


def _body(x_ref, wdkv_ref, wuk_ref, wuv_ref, wq_ref, wqr_ref, wkr_ref,
          wo_hbm, out_ref,
          kvown, csend, crecv, wsend, wrecv, q_scr, qr_scr, kr_scr,
          obuf, wobuf,
          c_sems, w_sems, ring_send_sems, ring_recv_sems, wo_sems):
    my_x = lax.axis_index("x")
    my_y = lax.axis_index("y")
    p = _ring_pos(my_x, my_y)
    pp = _ring_pos(1 - my_x, my_y)
    right = _ring_coords((p + 1) % 4)
    left = _ring_coords((p + 3) % 4)

    origin0 = p
    origin1 = (p + 3) % 4
    wo_fetch0 = pltpu.make_async_copy(
        wo_hbm.at[pl.ds(origin0 * CW, CW), :], wobuf.at[0], wo_sems.at[0])
    wo_fetch1 = pltpu.make_async_copy(
        wo_hbm.at[pl.ds(origin1 * CW, CW), :], wobuf.at[1], wo_sems.at[1])
    wo_fetch0.start()
    wo_fetch1.start()

    csend[...] = jnp.dot(x_ref[...], wdkv_ref[...], preferred_element_type=F32)
    wsend[0] = wuk_ref[:, pl.ds(pp * CW, CW)]
    wsend[1] = wuv_ref[:, pl.ds(pp * CW, CW)]

    barrier = pltpu.get_barrier_semaphore()
    for nbr in (right, left):
        pl.semaphore_signal(barrier, inc=1, device_id=nbr,
                            device_id_type=_MESH)
    pl.semaphore_wait(barrier, 2)

    c_rdma = pltpu.make_async_remote_copy(
        src_ref=csend, dst_ref=crecv,
        send_sem=c_sems.at[0], recv_sem=c_sems.at[1],
        device_id=(1 - my_x, my_y), device_id_type=_MESH)
    w_rdma = pltpu.make_async_remote_copy(
        src_ref=wsend, dst_ref=wrecv,
        send_sem=w_sems.at[0], recv_sem=w_sems.at[1],
        device_id=(1 - my_x, my_y), device_id_type=_MESH)
    c_rdma.start()
    w_rdma.start()

    x = x_ref[...]
    q_scr[...] = jnp.dot(x, wq_ref[...], preferred_element_type=F32)
    qr_scr[...] = jnp.dot(x, wqr_ref[...], preferred_element_type=F32)
    kr_scr[...] = jnp.dot(x, wkr_ref[...], preferred_element_type=F32)

    c_rdma.wait()
    w_rdma.wait()
    kvown[0] = (jnp.dot(csend[...], wuk_ref[:, pl.ds(p * CW, CW)],
                        preferred_element_type=F32)
                + jnp.dot(crecv[...], wrecv[0], preferred_element_type=F32))
    kvown[1] = (jnp.dot(csend[...], wuv_ref[:, pl.ds(p * CW, CW)],
                        preferred_element_type=F32)
                + jnp.dot(crecv[...], wrecv[1], preferred_element_type=F32))

    dims = (((1,), (1,)), ((), ()))
    ring = [[None, None], [None, None], [None, None]]

    def _hop(h, half):
        rows = pl.ds(half * S, S)
        r = pltpu.make_async_remote_copy(
            src_ref=obuf.at[h, rows, :], dst_ref=obuf.at[h + 1, rows, :],
            send_sem=ring_send_sems.at[2 * h + half],
            recv_sem=ring_recv_sems.at[2 * h + half],
            device_id=right, device_id_type=_MESH)
        r.start()
        ring[h][half] = r

    for b in range(B):
        rows = slice(b * S, (b + 1) * S)
        kr_b = kr_scr[rows, :]
        qr_b = qr_scr[rows, :]
        for j in range(HG):
            cols = slice(j * Dh, (j + 1) * Dh)
            q_bh = q_scr[rows, cols]
            k_bh = kvown[0, rows, cols]
            qr_bh = qr_b[:, j * Dr:(j + 1) * Dr]
            s = (lax.dot_general(q_bh, k_bh, dims, preferred_element_type=F32)
                 + lax.dot_general(qr_bh, kr_b, dims,
                                   preferred_element_type=F32)) * SCALE
            m = jnp.max(s, axis=-1, keepdims=True)
            pr = jnp.exp(s - m)
            pr = pr / jnp.sum(pr, axis=-1, keepdims=True)
            obuf[0, rows, cols] = jnp.dot(pr, kvown[1, rows, cols],
                                          preferred_element_type=F32)
        _hop(0, b)

    wo_fetch0.wait()
    out_ref[...] = jnp.dot(obuf[0], wobuf[0], preferred_element_type=F32)
    origin2 = (p + 2) % 4
    wo_fetch2 = pltpu.make_async_copy(
        wo_hbm.at[pl.ds(origin2 * CW, CW), :], wobuf.at[0], wo_sems.at[0])
    wo_fetch2.start()

    for h in range(1, 3):
        for half in range(B):
            ring[h - 1][half].wait()
            _hop(h, half)
        wo_fetchN = wo_fetch1 if h == 1 else wo_fetch2
        wo_fetchN.wait()
        out_ref[...] = out_ref[...] + jnp.dot(
            obuf[h], wobuf[h % 2], preferred_element_type=F32)
        if h == 1:
            origin3 = (p + 1) % 4
            wo_fetch3 = pltpu.make_async_copy(
                wo_hbm.at[pl.ds(origin3 * CW, CW), :], wobuf.at[1],
                wo_sems.at[1])
            wo_fetch3.start()

    ring[2][0].wait()
    ring[2][1].wait()
    wo_fetch3.wait()
    out_ref[...] = out_ref[...] + jnp.dot(
        obuf[3], wobuf[1], preferred_element_type=F32)


def kernel(x, Wdkv, Wuk, Wuv, Wq, Wqr, Wkr, Wo):
    x2d = x.reshape(BS, D)
    my_x = lax.axis_index("x")
    my_y = lax.axis_index("y")
    p = _ring_pos(my_x, my_y)
    wq_s = lax.dynamic_slice(Wq, (0, p * CW), (D, CW))
    wqr_s = lax.dynamic_slice(Wqr, (0, p * RW), (D, RW))

    out2d = pl.pallas_call(
        _body,
        out_shape=jax.ShapeDtypeStruct((BS, D), F32),
        in_specs=[pl.BlockSpec(memory_space=pltpu.VMEM)] * 7
        + [pl.BlockSpec(memory_space=pl.ANY)],
        out_specs=pl.BlockSpec(memory_space=pltpu.VMEM),
        scratch_shapes=[
            pltpu.VMEM((2, BS, CW), F32),
            pltpu.VMEM((BS, DC), F32),
            pltpu.VMEM((BS, DC), F32),
            pltpu.VMEM((2, DC, CW), F32),
            pltpu.VMEM((2, DC, CW), F32),
            pltpu.VMEM((BS, CW), F32),
            pltpu.VMEM((BS, RW), F32),
            pltpu.VMEM((BS, Dr), F32),
            pltpu.VMEM((4, BS, CW), F32),
            pltpu.VMEM((2, CW, D), F32),
            pltpu.SemaphoreType.DMA((2,)),
            pltpu.SemaphoreType.DMA((2,)),
            pltpu.SemaphoreType.DMA((6,)),
            pltpu.SemaphoreType.DMA((6,)),
            pltpu.SemaphoreType.DMA((2,)),
        ],
        compiler_params=pltpu.CompilerParams(
            collective_id=0, vmem_limit_bytes=60 * 1024 * 1024),
    )(x2d, Wdkv, Wuk, Wuv, wq_s, wqr_s, Wkr, Wo)
    return out2d.reshape(B, S, D)


# device time: 109204 ns/iter; 2.6421x vs baseline; 1.1152x over previous
import jax
import jax.numpy as jnp
from jax import lax
from jax.experimental import pallas as pl
from jax.experimental.pallas import tpu as pltpu

B, S, D = 2, 512, 2048
H, Dh, Dr = 16, 128, 32
DC = 128
BS = B * S
HG = H // 4
CW = HG * Dh
RW = HG * Dr
SCALE = (Dh + Dr) ** -0.5
F32 = jnp.float32

_MESH = pl.DeviceIdType.MESH


def _ring_pos(x, y):
    return 2 * x + (x ^ y)


def _ring_coords(q):
    return (q // 2, (q ^ (q // 2)) & 1)


def _body(x_ref, wdkv_ref, wuk_ref, wuv_ref, wq_hbm, wqr_hbm, wkr_ref,
          wo_hbm, out_ref,
          kvown, csend, crecv, wsend, wrecv, wqbuf, wqrbuf,
          q_scr, qr_scr, kr_scr, obuf, wobuf,
          c_sems, w_sems, ring_send_sems, ring_recv_sems, wo_sems, wq_sems):
    my_x = lax.axis_index("x")
    my_y = lax.axis_index("y")
    p = _ring_pos(my_x, my_y)
    pp = _ring_pos(1 - my_x, my_y)
    right = _ring_coords((p + 1) % 4)
    left = _ring_coords((p + 3) % 4)

    wq_fetch = pltpu.make_async_copy(
        wq_hbm.at[:, pl.ds(p * CW, CW)], wqbuf, wq_sems.at[0])
    wqr_fetch = pltpu.make_async_copy(
        wqr_hbm.at[:, pl.ds(p * RW, RW)], wqrbuf, wq_sems.at[1])
    wq_fetch.start()
    wqr_fetch.start()
    origin0 = p
    origin1 = (p + 3) % 4
    wo_fetch0 = pltpu.make_async_copy(
        wo_hbm.at[pl.ds(origin0 * CW, CW), :], wobuf.at[0], wo_sems.at[0])
    wo_fetch1 = pltpu.make_async_copy(
        wo_hbm.at[pl.ds(origin1 * CW, CW), :], wobuf.at[1], wo_sems.at[1])
    wo_fetch0.start()
    wo_fetch1.start()

    csend[...] = jnp.dot(x_ref[...], wdkv_ref[...], preferred_element_type=F32)
    wsend[0] = wuk_ref[:, pl.ds(pp * CW, CW)]
    wsend[1] = wuv_ref[:, pl.ds(pp * CW, CW)]

    barrier = pltpu.get_barrier_semaphore()
    for nbr in (right, left):
        pl.semaphore_signal(barrier, inc=1, device_id=nbr,
                            device_id_type=_MESH)
    pl.semaphore_wait(barrier, 2)

    c_rdma = pltpu.make_async_remote_copy(
        src_ref=csend, dst_ref=crecv,
        send_sem=c_sems.at[0], recv_sem=c_sems.at[1],
        device_id=(1 - my_x, my_y), device_id_type=_MESH)
    w_rdma = pltpu.make_async_remote_copy(
        src_ref=wsend, dst_ref=wrecv,
        send_sem=w_sems.at[0], recv_sem=w_sems.at[1],
        device_id=(1 - my_x, my_y), device_id_type=_MESH)
    c_rdma.start()
    w_rdma.start()

    x = x_ref[...]
    wq_fetch.wait()
    q_scr[...] = jnp.dot(x, wqbuf[...], preferred_element_type=F32)
    wqr_fetch.wait()
    qr_scr[...] = jnp.dot(x, wqrbuf[...], preferred_element_type=F32)
    kr_scr[...] = jnp.dot(x, wkr_ref[...], preferred_element_type=F32)
    kvown[0] = jnp.dot(csend[...], wuk_ref[:, pl.ds(p * CW, CW)],
                       preferred_element_type=F32)
    kvown[1] = jnp.dot(csend[...], wuv_ref[:, pl.ds(p * CW, CW)],
                       preferred_element_type=F32)

    c_rdma.wait()
    w_rdma.wait()
    kvown[0] = kvown[0] + jnp.dot(crecv[...], wrecv[0],
                                  preferred_element_type=F32)
    kvown[1] = kvown[1] + jnp.dot(crecv[...], wrecv[1],
                                  preferred_element_type=F32)

    dims = (((1,), (1,)), ((), ()))
    ring = [[None, None], [None, None], [None, None]]

    def _hop(h, half):
        rows = pl.ds(half * S, S)
        r = pltpu.make_async_remote_copy(
            src_ref=obuf.at[h, rows, :], dst_ref=obuf.at[h + 1, rows, :],
            send_sem=ring_send_sems.at[2 * h + half],
            recv_sem=ring_recv_sems.at[2 * h + half],
            device_id=right, device_id_type=_MESH)
        r.start()
        ring[h][half] = r

    for b in range(B):
        rows = slice(b * S, (b + 1) * S)
        kr_b = kr_scr[rows, :]
        qr_b = qr_scr[rows, :]
        for j in range(HG):
            cols = slice(j * Dh, (j + 1) * Dh)
            q_bh = q_scr[rows, cols]
            k_bh = kvown[0, rows, cols]
            qr_bh = qr_b[:, j * Dr:(j + 1) * Dr]
            s = (lax.dot_general(q_bh, k_bh, dims, preferred_element_type=F32)
                 + lax.dot_general(qr_bh, kr_b, dims,
                                   preferred_element_type=F32)) * SCALE
            m = jnp.max(s, axis=-1, keepdims=True)
            pr = jnp.exp(s - m)
            pr = pr / jnp.sum(pr, axis=-1, keepdims=True)
            obuf[0, rows, cols] = jnp.dot(pr, kvown[1, rows, cols],
                                          preferred_element_type=F32)
        _hop(0, b)

    wo_fetch0.wait()
    out_ref[...] = jnp.dot(obuf[0], wobuf[0], preferred_element_type=F32)
    origin2 = (p + 2) % 4
    wo_fetch2 = pltpu.make_async_copy(
        wo_hbm.at[pl.ds(origin2 * CW, CW), :], wobuf.at[0], wo_sems.at[0])
    wo_fetch2.start()

    for h in range(1, 3):
        wo_fetchN = wo_fetch1 if h == 1 else wo_fetch2
        for half in range(B):
            ring[h - 1][half].wait()
            _hop(h, half)
            if half == 0:
                wo_fetchN.wait()
            rows = slice(half * S, (half + 1) * S)
            out_ref[rows, :] = out_ref[rows, :] + jnp.dot(
                obuf[h, rows, :], wobuf[h % 2], preferred_element_type=F32)
        if h == 1:
            origin3 = (p + 1) % 4
            wo_fetch3 = pltpu.make_async_copy(
                wo_hbm.at[pl.ds(origin3 * CW, CW), :], wobuf.at[1],
                wo_sems.at[1])
            wo_fetch3.start()

    for half in range(B):
        ring[2][half].wait()
        if half == 0:
            wo_fetch3.wait()
        rows = slice(half * S, (half + 1) * S)
        out_ref[rows, :] = out_ref[rows, :] + jnp.dot(
            obuf[3, rows, :], wobuf[1], preferred_element_type=F32)


def kernel(x, Wdkv, Wuk, Wuv, Wq, Wqr, Wkr, Wo):
    x2d = x.reshape(BS, D)

    out2d = pl.pallas_call(
        _body,
        out_shape=jax.ShapeDtypeStruct((BS, D), F32),
        in_specs=[pl.BlockSpec(memory_space=pltpu.VMEM)] * 4
        + [pl.BlockSpec(memory_space=pl.ANY)] * 2
        + [pl.BlockSpec(memory_space=pltpu.VMEM)]
        + [pl.BlockSpec(memory_space=pl.ANY)],
        out_specs=pl.BlockSpec(memory_space=pltpu.VMEM),
        scratch_shapes=[
            pltpu.VMEM((2, BS, CW), F32),
            pltpu.VMEM((BS, DC), F32),
            pltpu.VMEM((BS, DC), F32),
            pltpu.VMEM((2, DC, CW), F32),
            pltpu.VMEM((2, DC, CW), F32),
            pltpu.VMEM((D, CW), F32),
            pltpu.VMEM((D, RW), F32),
            pltpu.VMEM((BS, CW), F32),
            pltpu.VMEM((BS, RW), F32),
            pltpu.VMEM((BS, Dr), F32),
            pltpu.VMEM((4, BS, CW), F32),
            pltpu.VMEM((2, CW, D), F32),
            pltpu.SemaphoreType.DMA((2,)),
            pltpu.SemaphoreType.DMA((2,)),
            pltpu.SemaphoreType.DMA((6,)),
            pltpu.SemaphoreType.DMA((6,)),
            pltpu.SemaphoreType.DMA((2,)),
            pltpu.SemaphoreType.DMA((2,)),
        ],
        compiler_params=pltpu.CompilerParams(
            collective_id=0, vmem_limit_bytes=60 * 1024 * 1024),
    )(x2d, Wdkv, Wuk, Wuv, Wq, Wqr, Wkr, Wo)
    return out2d.reshape(B, S, D)


# device time: 105558 ns/iter; 2.7334x vs baseline; 1.0345x over previous
import jax
import jax.numpy as jnp
from jax import lax
from jax.experimental import pallas as pl
from jax.experimental.pallas import tpu as pltpu

B, S, D = 2, 512, 2048
H, Dh, Dr = 16, 128, 32
DC = 128
BS = B * S
HG = H // 4
CW = HG * Dh
RW = HG * Dr
SCALE = (Dh + Dr) ** -0.5
F32 = jnp.float32

_MESH = pl.DeviceIdType.MESH


def _ring_pos(x, y):
    return 2 * x + (x ^ y)


def _ring_coords(q):
    return (q // 2, (q ^ (q // 2)) & 1)


def _body(x_ref, wdkv_ref, wuk_ref, wuv_ref, wq_hbm, wqr_hbm, wkr_ref,
          wo_hbm, out_ref,
          kvown, csend, crecv, wsend, wrecv, wqbuf, wqrbuf,
          q_scr, qr_scr, kr_scr, obuf, wobuf,
          c_sems, w_sems, ring_send_sems, ring_recv_sems, wo_sems, wq_sems):
    my_x = lax.axis_index("x")
    my_y = lax.axis_index("y")
    p = _ring_pos(my_x, my_y)
    pp = _ring_pos(1 - my_x, my_y)
    right = _ring_coords((p + 1) % 4)
    left = _ring_coords((p + 3) % 4)

    wq_fetch = pltpu.make_async_copy(
        wq_hbm.at[:, pl.ds(p * CW, CW)], wqbuf, wq_sems.at[0])
    wqr_fetch = pltpu.make_async_copy(
        wqr_hbm.at[:, pl.ds(p * RW, RW)], wqrbuf, wq_sems.at[1])
    wq_fetch.start()
    wqr_fetch.start()
    origin0 = p
    origin1 = (p + 3) % 4
    wo_fetch0 = pltpu.make_async_copy(
        wo_hbm.at[pl.ds(origin0 * CW, CW), :], wobuf.at[0], wo_sems.at[0])
    wo_fetch1 = pltpu.make_async_copy(
        wo_hbm.at[pl.ds(origin1 * CW, CW), :], wobuf.at[1], wo_sems.at[1])
    wo_fetch0.start()
    wo_fetch1.start()

    csend[...] = jnp.dot(x_ref[...], wdkv_ref[...], preferred_element_type=F32)
    wsend[0] = wuk_ref[:, pl.ds(pp * CW, CW)]
    wsend[1] = wuv_ref[:, pl.ds(pp * CW, CW)]

    barrier = pltpu.get_barrier_semaphore()
    for nbr in (right, left):
        pl.semaphore_signal(barrier, inc=1, device_id=nbr,
                            device_id_type=_MESH)
    pl.semaphore_wait(barrier, 2)

    w_rdma = pltpu.make_async_remote_copy(
        src_ref=wsend, dst_ref=wrecv,
        send_sem=w_sems.at[0], recv_sem=w_sems.at[1],
        device_id=(1 - my_x, my_y), device_id_type=_MESH)
    w_rdma.start()
    c_rdmas = []
    for b in range(B):
        rows = pl.ds(b * S, S)
        r = pltpu.make_async_remote_copy(
            src_ref=csend.at[rows, :], dst_ref=crecv.at[rows, :],
            send_sem=c_sems.at[2 * b], recv_sem=c_sems.at[2 * b + 1],
            device_id=(1 - my_x, my_y), device_id_type=_MESH)
        r.start()
        c_rdmas.append(r)

    x = x_ref[...]
    wq_fetch.wait()
    wqr_fetch.wait()
    dims = (((1,), (1,)), ((), ()))
    ring = [[None, None], [None, None], [None, None]]

    def _hop(h, half):
        rows = pl.ds(half * S, S)
        r = pltpu.make_async_remote_copy(
            src_ref=obuf.at[h, rows, :], dst_ref=obuf.at[h + 1, rows, :],
            send_sem=ring_send_sems.at[2 * h + half],
            recv_sem=ring_recv_sems.at[2 * h + half],
            device_id=right, device_id_type=_MESH)
        r.start()
        ring[h][half] = r

    for b in range(B):
        rows = slice(b * S, (b + 1) * S)
        x_b = x[rows, :]
        q_scr[rows, :] = jnp.dot(x_b, wqbuf[...], preferred_element_type=F32)
        qr_scr[rows, :] = jnp.dot(x_b, wqrbuf[...], preferred_element_type=F32)
        kr_scr[rows, :] = jnp.dot(x_b, wkr_ref[...], preferred_element_type=F32)
        kvown[0, rows, :] = jnp.dot(csend[rows, :],
                                    wuk_ref[:, pl.ds(p * CW, CW)],
                                    preferred_element_type=F32)
        kvown[1, rows, :] = jnp.dot(csend[rows, :],
                                    wuv_ref[:, pl.ds(p * CW, CW)],
                                    preferred_element_type=F32)
        if b == 0:
            w_rdma.wait()
        c_rdmas[b].wait()
        kvown[0, rows, :] = kvown[0, rows, :] + jnp.dot(
            crecv[rows, :], wrecv[0], preferred_element_type=F32)
        kvown[1, rows, :] = kvown[1, rows, :] + jnp.dot(
            crecv[rows, :], wrecv[1], preferred_element_type=F32)

        kr_b = kr_scr[rows, :]
        qr_b = qr_scr[rows, :]
        for j in range(HG):
            cols = slice(j * Dh, (j + 1) * Dh)
            q_bh = q_scr[rows, cols]
            k_bh = kvown[0, rows, cols]
            qr_bh = qr_b[:, j * Dr:(j + 1) * Dr]
            s = (lax.dot_general(q_bh, k_bh, dims, preferred_element_type=F32)
                 + lax.dot_general(qr_bh, kr_b, dims,
                                   preferred_element_type=F32)) * SCALE
            m = jnp.max(s, axis=-1, keepdims=True)
            pr = jnp.exp(s - m)
            pr = pr / jnp.sum(pr, axis=-1, keepdims=True)
            obuf[0, rows, cols] = jnp.dot(pr, kvown[1, rows, cols],
                                          preferred_element_type=F32)
        _hop(0, b)

    wo_fetch0.wait()
    out_ref[...] = jnp.dot(obuf[0], wobuf[0], preferred_element_type=F32)
    origin2 = (p + 2) % 4
    wo_fetch2 = pltpu.make_async_copy(
        wo_hbm.at[pl.ds(origin2 * CW, CW), :], wobuf.at[0], wo_sems.at[0])
    wo_fetch2.start()

    for h in range(1, 3):
        wo_fetchN = wo_fetch1 if h == 1 else wo_fetch2
        for half in range(B):
            ring[h - 1][half].wait()
            _hop(h, half)
            if half == 0:
                wo_fetchN.wait()
            rows = slice(half * S, (half + 1) * S)
            out_ref[rows, :] = out_ref[rows, :] + jnp.dot(
                obuf[h, rows, :], wobuf[h % 2], preferred_element_type=F32)
        if h == 1:
            origin3 = (p + 1) % 4
            wo_fetch3 = pltpu.make_async_copy(
                wo_hbm.at[pl.ds(origin3 * CW, CW), :], wobuf.at[1],
                wo_sems.at[1])
            wo_fetch3.start()

    for half in range(B):
        ring[2][half].wait()
        if half == 0:
            wo_fetch3.wait()
        rows = slice(half * S, (half + 1) * S)
        out_ref[rows, :] = out_ref[rows, :] + jnp.dot(
            obuf[3, rows, :], wobuf[1], preferred_element_type=F32)


def kernel(x, Wdkv, Wuk, Wuv, Wq, Wqr, Wkr, Wo):
    x2d = x.reshape(BS, D)

    out2d = pl.pallas_call(
        _body,
        out_shape=jax.ShapeDtypeStruct((BS, D), F32),
        in_specs=[pl.BlockSpec(memory_space=pltpu.VMEM)] * 4
        + [pl.BlockSpec(memory_space=pl.ANY)] * 2
        + [pl.BlockSpec(memory_space=pltpu.VMEM)]
        + [pl.BlockSpec(memory_space=pl.ANY)],
        out_specs=pl.BlockSpec(memory_space=pltpu.VMEM),
        scratch_shapes=[
            pltpu.VMEM((2, BS, CW), F32),
            pltpu.VMEM((BS, DC), F32),
            pltpu.VMEM((BS, DC), F32),
            pltpu.VMEM((2, DC, CW), F32),
            pltpu.VMEM((2, DC, CW), F32),
            pltpu.VMEM((D, CW), F32),
            pltpu.VMEM((D, RW), F32),
            pltpu.VMEM((BS, CW), F32),
            pltpu.VMEM((BS, RW), F32),
            pltpu.VMEM((BS, Dr), F32),
            pltpu.VMEM((4, BS, CW), F32),
            pltpu.VMEM((2, CW, D), F32),
            pltpu.SemaphoreType.DMA((4,)),
            pltpu.SemaphoreType.DMA((2,)),
            pltpu.SemaphoreType.DMA((6,)),
            pltpu.SemaphoreType.DMA((6,)),
            pltpu.SemaphoreType.DMA((2,)),
            pltpu.SemaphoreType.DMA((2,)),
        ],
        compiler_params=pltpu.CompilerParams(
            collective_id=0, vmem_limit_bytes=60 * 1024 * 1024),
    )(x2d, Wdkv, Wuk, Wuv, Wq, Wqr, Wkr, Wo)
    return out2d.reshape(B, S, D)


# device time: 70207 ns/iter; 4.1097x vs baseline; 1.5035x over previous
import jax
import jax.numpy as jnp
from jax import lax
from jax.experimental import pallas as pl
from jax.experimental.pallas import tpu as pltpu

B, S, D = 2, 512, 2048
H, Dh, Dr = 16, 128, 32
DC = 128
BS = B * S
HG = H // 4
CW = HG * Dh
RW = HG * Dr
SCALE = (Dh + Dr) ** -0.5
F32 = jnp.float32
BF16 = jnp.bfloat16

_MESH = pl.DeviceIdType.MESH


def _ring_pos(x, y):
    return 2 * x + (x ^ y)


def _ring_coords(q):
    return (q // 2, (q ^ (q // 2)) & 1)


def _bdot(a, b):
    return lax.dot_general(a, b, (((1,), (0,)), ((), ())),
                           preferred_element_type=F32)


def _body(x_ref, wdkv_ref, wuk_ref, wuv_ref, wq_hbm, wqr_hbm, wkr_ref,
          wo_hbm, out_ref,
          kvown, csend, crecv, wsend, wrecv, wqbuf, wqrbuf,
          q_scr, qr_scr, kr_scr, obuf, wobuf,
          c_sems, w_sems, ring_send_sems, ring_recv_sems, wo_sems, wq_sems):
    my_x = lax.axis_index("x")
    my_y = lax.axis_index("y")
    p = _ring_pos(my_x, my_y)
    pp = _ring_pos(1 - my_x, my_y)
    right = _ring_coords((p + 1) % 4)
    left = _ring_coords((p + 3) % 4)

    wq_fetch = pltpu.make_async_copy(
        wq_hbm.at[:, pl.ds(p * CW, CW)], wqbuf, wq_sems.at[0])
    wqr_fetch = pltpu.make_async_copy(
        wqr_hbm.at[:, pl.ds(p * RW, RW)], wqrbuf, wq_sems.at[1])
    wq_fetch.start()
    wqr_fetch.start()
    origin0 = p
    origin1 = (p + 3) % 4
    wo_fetch0 = pltpu.make_async_copy(
        wo_hbm.at[pl.ds(origin0 * CW, CW), :], wobuf.at[0], wo_sems.at[0])
    wo_fetch1 = pltpu.make_async_copy(
        wo_hbm.at[pl.ds(origin1 * CW, CW), :], wobuf.at[1], wo_sems.at[1])
    wo_fetch0.start()
    wo_fetch1.start()

    x_bf = x_ref[...].astype(BF16)
    csend[...] = _bdot(x_bf, wdkv_ref[...].astype(BF16)).astype(BF16)
    wsend[0] = wuk_ref[:, pl.ds(pp * CW, CW)].astype(BF16)
    wsend[1] = wuv_ref[:, pl.ds(pp * CW, CW)].astype(BF16)

    barrier = pltpu.get_barrier_semaphore()
    for nbr in (right, left):
        pl.semaphore_signal(barrier, inc=1, device_id=nbr,
                            device_id_type=_MESH)
    pl.semaphore_wait(barrier, 2)

    w_rdma = pltpu.make_async_remote_copy(
        src_ref=wsend, dst_ref=wrecv,
        send_sem=w_sems.at[0], recv_sem=w_sems.at[1],
        device_id=(1 - my_x, my_y), device_id_type=_MESH)
    w_rdma.start()
    c_rdmas = []
    for b in range(B):
        rows = pl.ds(b * S, S)
        r = pltpu.make_async_remote_copy(
            src_ref=csend.at[rows, :], dst_ref=crecv.at[rows, :],
            send_sem=c_sems.at[2 * b], recv_sem=c_sems.at[2 * b + 1],
            device_id=(1 - my_x, my_y), device_id_type=_MESH)
        r.start()
        c_rdmas.append(r)

    wq_fetch.wait()
    wqr_fetch.wait()
    wq_bf = wqbuf[...].astype(BF16)
    wqr_bf = wqrbuf[...].astype(BF16)
    wkr_bf = wkr_ref[...].astype(BF16)
    wuk_own = wuk_ref[:, pl.ds(p * CW, CW)].astype(BF16)
    wuv_own = wuv_ref[:, pl.ds(p * CW, CW)].astype(BF16)
    dims_t = (((1,), (1,)), ((), ()))
    ring = [[None, None], [None, None], [None, None]]

    def _hop(h, half):
        rows = pl.ds(half * S, S)
        r = pltpu.make_async_remote_copy(
            src_ref=obuf.at[h, rows, :], dst_ref=obuf.at[h + 1, rows, :],
            send_sem=ring_send_sems.at[2 * h + half],
            recv_sem=ring_recv_sems.at[2 * h + half],
            device_id=right, device_id_type=_MESH)
        r.start()
        ring[h][half] = r

    for b in range(B):
        rows = slice(b * S, (b + 1) * S)
        x_b = x_bf[rows, :]
        q_scr[rows, :] = _bdot(x_b, wq_bf).astype(BF16)
        qr_scr[rows, :] = _bdot(x_b, wqr_bf).astype(BF16)
        kr_scr[rows, :] = _bdot(x_b, wkr_bf).astype(BF16)
        k_part = _bdot(csend[rows, :], wuk_own)
        v_part = _bdot(csend[rows, :], wuv_own)
        if b == 0:
            w_rdma.wait()
        c_rdmas[b].wait()
        kvown[0, rows, :] = (k_part + _bdot(crecv[rows, :], wrecv[0])
                             ).astype(BF16)
        kvown[1, rows, :] = (v_part + _bdot(crecv[rows, :], wrecv[1])
                             ).astype(BF16)

        kr_b = kr_scr[rows, :]
        qr_b = qr_scr[rows, :]
        for j in range(HG):
            cols = slice(j * Dh, (j + 1) * Dh)
            q_bh = q_scr[rows, cols]
            k_bh = kvown[0, rows, cols]
            qr_bh = qr_b[:, j * Dr:(j + 1) * Dr]
            s = (lax.dot_general(q_bh, k_bh, dims_t,
                                 preferred_element_type=F32)
                 + lax.dot_general(qr_bh, kr_b, dims_t,
                                   preferred_element_type=F32)) * SCALE
            m = jnp.max(s, axis=-1, keepdims=True)
            pr = jnp.exp(s - m)
            pr = (pr / jnp.sum(pr, axis=-1, keepdims=True)).astype(BF16)
            obuf[0, rows, cols] = _bdot(pr, kvown[1, rows, cols]
                                        ).astype(BF16)
        _hop(0, b)

    wo_fetch0.wait()
    out_ref[...] = _bdot(obuf[0], wobuf[0].astype(BF16))
    origin2 = (p + 2) % 4
    wo_fetch2 = pltpu.make_async_copy(
        wo_hbm.at[pl.ds(origin2 * CW, CW), :], wobuf.at[0], wo_sems.at[0])
    wo_fetch2.start()

    for h in range(1, 3):
        wo_fetchN = wo_fetch1 if h == 1 else wo_fetch2
        for half in range(B):
            ring[h - 1][half].wait()
            _hop(h, half)
            if half == 0:
                wo_fetchN.wait()
            rows = slice(half * S, (half + 1) * S)
            out_ref[rows, :] = out_ref[rows, :] + _bdot(
                obuf[h, rows, :], wobuf[h % 2].astype(BF16))
        if h == 1:
            origin3 = (p + 1) % 4
            wo_fetch3 = pltpu.make_async_copy(
                wo_hbm.at[pl.ds(origin3 * CW, CW), :], wobuf.at[1],
                wo_sems.at[1])
            wo_fetch3.start()

    for half in range(B):
        ring[2][half].wait()
        if half == 0:
            wo_fetch3.wait()
        rows = slice(half * S, (half + 1) * S)
        out_ref[rows, :] = out_ref[rows, :] + _bdot(
            obuf[3, rows, :], wobuf[1].astype(BF16))


def kernel(x, Wdkv, Wuk, Wuv, Wq, Wqr, Wkr, Wo):
    x2d = x.reshape(BS, D)

    out2d = pl.pallas_call(
        _body,
        out_shape=jax.ShapeDtypeStruct((BS, D), F32),
        in_specs=[pl.BlockSpec(memory_space=pltpu.VMEM)] * 4
        + [pl.BlockSpec(memory_space=pl.ANY)] * 2
        + [pl.BlockSpec(memory_space=pltpu.VMEM)]
        + [pl.BlockSpec(memory_space=pl.ANY)],
        out_specs=pl.BlockSpec(memory_space=pltpu.VMEM),
        scratch_shapes=[
            pltpu.VMEM((2, BS, CW), BF16),
            pltpu.VMEM((BS, DC), BF16),
            pltpu.VMEM((BS, DC), BF16),
            pltpu.VMEM((2, DC, CW), BF16),
            pltpu.VMEM((2, DC, CW), BF16),
            pltpu.VMEM((D, CW), F32),
            pltpu.VMEM((D, RW), F32),
            pltpu.VMEM((BS, CW), BF16),
            pltpu.VMEM((BS, RW), BF16),
            pltpu.VMEM((BS, Dr), BF16),
            pltpu.VMEM((4, BS, CW), BF16),
            pltpu.VMEM((2, CW, D), F32),
            pltpu.SemaphoreType.DMA((4,)),
            pltpu.SemaphoreType.DMA((2,)),
            pltpu.SemaphoreType.DMA((6,)),
            pltpu.SemaphoreType.DMA((6,)),
            pltpu.SemaphoreType.DMA((2,)),
            pltpu.SemaphoreType.DMA((2,)),
        ],
        compiler_params=pltpu.CompilerParams(
            collective_id=0, vmem_limit_bytes=60 * 1024 * 1024),
    )(x2d, Wdkv, Wuk, Wuv, Wq, Wqr, Wkr, Wo)
    return out2d.reshape(B, S, D)


# device time: 61216 ns/iter; 4.7134x vs baseline; 1.1469x over previous
import jax
import jax.numpy as jnp
from jax import lax
from jax.experimental import pallas as pl
from jax.experimental.pallas import tpu as pltpu

B, S, D = 2, 512, 2048
H, Dh, Dr = 16, 128, 32
DC = 128
BS = B * S
HG = H // 4
CW = HG * Dh
RW = HG * Dr
SCALE = (Dh + Dr) ** -0.5
F32 = jnp.float32
BF16 = jnp.bfloat16

_MESH = pl.DeviceIdType.MESH


def _ring_pos(x, y):
    return 2 * x + (x ^ y)


def _ring_coords(q):
    return (q // 2, (q ^ (q // 2)) & 1)


def _bdot(a, b):
    return lax.dot_general(a, b, (((1,), (0,)), ((), ())),
                           preferred_element_type=F32)


def _body(x_ref, wdkv_ref, wuk_ref, wuv_ref, wq_hbm, wqr_hbm, wkr_ref,
          wo_hbm, out_ref,
          kvown, csend, crecv, wsend, wrecv, wqbuf, wqrbuf,
          q_scr, qr_scr, kr_scr, obuf, wobuf,
          c_sems, w_sems, ring_send_sems, ring_recv_sems, wo_sems, wq_sems):
    my_x = lax.axis_index("x")
    my_y = lax.axis_index("y")
    p = _ring_pos(my_x, my_y)
    pp = _ring_pos(1 - my_x, my_y)
    right = _ring_coords((p + 1) % 4)
    left = _ring_coords((p + 3) % 4)

    wq_fetch = pltpu.make_async_copy(
        wq_hbm.at[:, pl.ds(p * CW, CW)], wqbuf, wq_sems.at[0])
    wqr_fetch = pltpu.make_async_copy(
        wqr_hbm.at[:, pl.ds(p * RW, RW)], wqrbuf, wq_sems.at[1])
    wq_fetch.start()
    wqr_fetch.start()
    origin0 = p
    origin1 = (p + 3) % 4
    wo_fetch0 = pltpu.make_async_copy(
        wo_hbm.at[pl.ds(origin0 * CW, CW), :], wobuf.at[0], wo_sems.at[0])
    wo_fetch1 = pltpu.make_async_copy(
        wo_hbm.at[pl.ds(origin1 * CW, CW), :], wobuf.at[1], wo_sems.at[1])
    wo_fetch0.start()
    wo_fetch1.start()

    x_bf = x_ref[...].astype(BF16)
    csend[...] = _bdot(x_bf, wdkv_ref[...].astype(BF16)).astype(BF16)
    wsend[0] = wuk_ref[:, pl.ds(pp * CW, CW)].astype(BF16)
    wsend[1] = wuv_ref[:, pl.ds(pp * CW, CW)].astype(BF16)

    barrier = pltpu.get_barrier_semaphore()
    for k in range(1, 4):
        pl.semaphore_signal(barrier, inc=1,
                            device_id=_ring_coords((p + k) % 4),
                            device_id_type=_MESH)
    pl.semaphore_wait(barrier, 3)

    w_rdma = pltpu.make_async_remote_copy(
        src_ref=wsend, dst_ref=wrecv,
        send_sem=w_sems.at[0], recv_sem=w_sems.at[1],
        device_id=(1 - my_x, my_y), device_id_type=_MESH)
    w_rdma.start()
    c_rdmas = []
    for b in range(B):
        rows = pl.ds(b * S, S)
        r = pltpu.make_async_remote_copy(
            src_ref=csend.at[rows, :], dst_ref=crecv.at[rows, :],
            send_sem=c_sems.at[2 * b], recv_sem=c_sems.at[2 * b + 1],
            device_id=(1 - my_x, my_y), device_id_type=_MESH)
        r.start()
        c_rdmas.append(r)

    wq_fetch.wait()
    wqr_fetch.wait()
    wq_bf = wqbuf[...].astype(BF16)
    wqr_bf = wqrbuf[...].astype(BF16)
    wkr_bf = wkr_ref[...].astype(BF16)
    wuk_own = wuk_ref[:, pl.ds(p * CW, CW)].astype(BF16)
    wuv_own = wuv_ref[:, pl.ds(p * CW, CW)].astype(BF16)
    dims_t = (((1,), (1,)), ((), ()))
    ring = [[None, None], [None, None], [None, None]]

    def _send(half):
        rows = pl.ds(half * S, S)
        for k in range(1, 4):
            idx = 2 * (k - 1) + half
            r = pltpu.make_async_remote_copy(
                src_ref=obuf.at[0, rows, :], dst_ref=obuf.at[k, rows, :],
                send_sem=ring_send_sems.at[idx],
                recv_sem=ring_recv_sems.at[idx],
                device_id=_ring_coords((p + k) % 4), device_id_type=_MESH)
            r.start()
            ring[k - 1][half] = r

    for b in range(B):
        rows = slice(b * S, (b + 1) * S)
        x_b = x_bf[rows, :]
        q_scr[rows, :] = _bdot(x_b, wq_bf).astype(BF16)
        qr_scr[rows, :] = _bdot(x_b, wqr_bf).astype(BF16)
        kr_scr[rows, :] = _bdot(x_b, wkr_bf).astype(BF16)
        k_part = _bdot(csend[rows, :], wuk_own)
        v_part = _bdot(csend[rows, :], wuv_own)
        if b == 0:
            w_rdma.wait()
        c_rdmas[b].wait()
        kvown[0, rows, :] = (k_part + _bdot(crecv[rows, :], wrecv[0])
                             ).astype(BF16)
        kvown[1, rows, :] = (v_part + _bdot(crecv[rows, :], wrecv[1])
                             ).astype(BF16)

        kr_b = kr_scr[rows, :]
        qr_b = qr_scr[rows, :]
        for j in range(HG):
            cols = slice(j * Dh, (j + 1) * Dh)
            q_bh = q_scr[rows, cols]
            k_bh = kvown[0, rows, cols]
            qr_bh = qr_b[:, j * Dr:(j + 1) * Dr]
            s = (lax.dot_general(q_bh, k_bh, dims_t,
                                 preferred_element_type=F32)
                 + lax.dot_general(qr_bh, kr_b, dims_t,
                                   preferred_element_type=F32)) * SCALE
            m = jnp.max(s, axis=-1, keepdims=True)
            pr = jnp.exp(s - m)
            pr = (pr / jnp.sum(pr, axis=-1, keepdims=True)).astype(BF16)
            obuf[0, rows, cols] = _bdot(pr, kvown[1, rows, cols]
                                        ).astype(BF16)
        _send(b)

    wo_fetch0.wait()
    out_ref[...] = _bdot(obuf[0], wobuf[0].astype(BF16))
    origin2 = (p + 2) % 4
    wo_fetch2 = pltpu.make_async_copy(
        wo_hbm.at[pl.ds(origin2 * CW, CW), :], wobuf.at[0], wo_sems.at[0])
    wo_fetch2.start()

    for h in range(1, 3):
        wo_fetchN = wo_fetch1 if h == 1 else wo_fetch2
        for half in range(B):
            ring[h - 1][half].wait()
            if half == 0:
                wo_fetchN.wait()
            rows = slice(half * S, (half + 1) * S)
            out_ref[rows, :] = out_ref[rows, :] + _bdot(
                obuf[h, rows, :], wobuf[h % 2].astype(BF16))
        if h == 1:
            origin3 = (p + 1) % 4
            wo_fetch3 = pltpu.make_async_copy(
                wo_hbm.at[pl.ds(origin3 * CW, CW), :], wobuf.at[1],
                wo_sems.at[1])
            wo_fetch3.start()

    for half in range(B):
        ring[2][half].wait()
        if half == 0:
            wo_fetch3.wait()
        rows = slice(half * S, (half + 1) * S)
        out_ref[rows, :] = out_ref[rows, :] + _bdot(
            obuf[3, rows, :], wobuf[1].astype(BF16))


def kernel(x, Wdkv, Wuk, Wuv, Wq, Wqr, Wkr, Wo):
    x2d = x.reshape(BS, D)

    out2d = pl.pallas_call(
        _body,
        out_shape=jax.ShapeDtypeStruct((BS, D), F32),
        in_specs=[pl.BlockSpec(memory_space=pltpu.VMEM)] * 4
        + [pl.BlockSpec(memory_space=pl.ANY)] * 2
        + [pl.BlockSpec(memory_space=pltpu.VMEM)]
        + [pl.BlockSpec(memory_space=pl.ANY)],
        out_specs=pl.BlockSpec(memory_space=pltpu.VMEM),
        scratch_shapes=[
            pltpu.VMEM((2, BS, CW), BF16),
            pltpu.VMEM((BS, DC), BF16),
            pltpu.VMEM((BS, DC), BF16),
            pltpu.VMEM((2, DC, CW), BF16),
            pltpu.VMEM((2, DC, CW), BF16),
            pltpu.VMEM((D, CW), F32),
            pltpu.VMEM((D, RW), F32),
            pltpu.VMEM((BS, CW), BF16),
            pltpu.VMEM((BS, RW), BF16),
            pltpu.VMEM((BS, Dr), BF16),
            pltpu.VMEM((4, BS, CW), BF16),
            pltpu.VMEM((2, CW, D), F32),
            pltpu.SemaphoreType.DMA((4,)),
            pltpu.SemaphoreType.DMA((2,)),
            pltpu.SemaphoreType.DMA((6,)),
            pltpu.SemaphoreType.DMA((6,)),
            pltpu.SemaphoreType.DMA((2,)),
            pltpu.SemaphoreType.DMA((2,)),
        ],
        compiler_params=pltpu.CompilerParams(
            collective_id=0, vmem_limit_bytes=60 * 1024 * 1024),
    )(x2d, Wdkv, Wuk, Wuv, Wq, Wqr, Wkr, Wo)
    return out2d.reshape(B, S, D)


# device time: 60194 ns/iter; 4.7934x vs baseline; 1.0170x over previous
import jax
import jax.numpy as jnp
from jax import lax
from jax.experimental import pallas as pl
from jax.experimental.pallas import tpu as pltpu

B, S, D = 2, 512, 2048
H, Dh, Dr = 16, 128, 32
DC = 128
BS = B * S
HG = H // 4
CW = HG * Dh
RW = HG * Dr
SCALE = (Dh + Dr) ** -0.5
F32 = jnp.float32
BF16 = jnp.bfloat16

_MESH = pl.DeviceIdType.MESH


def _ring_pos(x, y):
    return 2 * x + (x ^ y)


def _ring_coords(q):
    return (q // 2, (q ^ (q // 2)) & 1)


def _bdot(a, b):
    return lax.dot_general(a, b, (((1,), (0,)), ((), ())),
                           preferred_element_type=F32)


def _body(x_ref, wdkv_ref, wuk_ref, wuv_ref, wq_hbm, wqr_hbm, wkr_ref,
          wo_hbm, out_ref,
          kvown, csend, crecv, wsend, wrecv, wqbuf, wqrbuf,
          q_scr, qr_scr, kr_scr, obuf, wobuf,
          c_sems, w_sems, ring_send_sems, ring_recv_sems, wo_sems, wq_sems):
    my_x = lax.axis_index("x")
    my_y = lax.axis_index("y")
    p = _ring_pos(my_x, my_y)
    pp = _ring_pos(1 - my_x, my_y)
    right = _ring_coords((p + 1) % 4)
    left = _ring_coords((p + 3) % 4)

    wq_fetch = pltpu.make_async_copy(
        wq_hbm.at[:, pl.ds(p * CW, CW)], wqbuf, wq_sems.at[0])
    wqr_fetch = pltpu.make_async_copy(
        wqr_hbm.at[:, pl.ds(p * RW, RW)], wqrbuf, wq_sems.at[1])
    wq_fetch.start()
    wqr_fetch.start()
    origin0 = p
    origin1 = (p + 3) % 4
    wo_fetch0 = pltpu.make_async_copy(
        wo_hbm.at[pl.ds(origin0 * CW, CW), :], wobuf.at[0], wo_sems.at[0])
    wo_fetch1 = pltpu.make_async_copy(
        wo_hbm.at[pl.ds(origin1 * CW, CW), :], wobuf.at[1], wo_sems.at[1])
    wo_fetch0.start()
    wo_fetch1.start()

    x_bf = x_ref[...].astype(BF16)
    csend[...] = _bdot(x_bf, wdkv_ref[...].astype(BF16)).astype(BF16)
    wsend[0] = wuk_ref[:, pl.ds(pp * CW, CW)].astype(BF16)
    wsend[1] = wuv_ref[:, pl.ds(pp * CW, CW)].astype(BF16)

    barrier = pltpu.get_barrier_semaphore()
    for k in range(1, 4):
        pl.semaphore_signal(barrier, inc=1,
                            device_id=_ring_coords((p + k) % 4),
                            device_id_type=_MESH)
    pl.semaphore_wait(barrier, 3)

    w_rdma = pltpu.make_async_remote_copy(
        src_ref=wsend, dst_ref=wrecv,
        send_sem=w_sems.at[0], recv_sem=w_sems.at[1],
        device_id=(1 - my_x, my_y), device_id_type=_MESH)
    w_rdma.start()
    c_rdmas = []
    for b in range(B):
        rows = pl.ds(b * S, S)
        r = pltpu.make_async_remote_copy(
            src_ref=csend.at[rows, :], dst_ref=crecv.at[rows, :],
            send_sem=c_sems.at[2 * b], recv_sem=c_sems.at[2 * b + 1],
            device_id=(1 - my_x, my_y), device_id_type=_MESH)
        r.start()
        c_rdmas.append(r)

    wq_fetch.wait()
    wqr_fetch.wait()
    wq_bf = wqbuf[...].astype(BF16)
    wqr_bf = wqrbuf[...].astype(BF16)
    wkr_bf = wkr_ref[...].astype(BF16)
    wuk_own = wuk_ref[:, pl.ds(p * CW, CW)].astype(BF16)
    wuv_own = wuv_ref[:, pl.ds(p * CW, CW)].astype(BF16)
    dims_t = (((1,), (1,)), ((), ()))
    ring = [[None, None], [None, None], [None, None]]

    def _send(half):
        rows = pl.ds(half * S, S)
        for k in range(1, 4):
            idx = 2 * (k - 1) + half
            r = pltpu.make_async_remote_copy(
                src_ref=obuf.at[0, rows, :], dst_ref=obuf.at[k, rows, :],
                send_sem=ring_send_sems.at[idx],
                recv_sem=ring_recv_sems.at[idx],
                device_id=_ring_coords((p + k) % 4), device_id_type=_MESH)
            r.start()
            ring[k - 1][half] = r

    for b in range(B):
        rows = slice(b * S, (b + 1) * S)
        x_b = x_bf[rows, :]
        q_scr[rows, :] = _bdot(x_b, wq_bf).astype(BF16)
        qr_scr[rows, :] = _bdot(x_b, wqr_bf).astype(BF16)
        kr_scr[rows, :] = _bdot(x_b, wkr_bf).astype(BF16)
        k_part = _bdot(csend[rows, :], wuk_own)
        v_part = _bdot(csend[rows, :], wuv_own)
        if b == 0:
            w_rdma.wait()
        c_rdmas[b].wait()
        kvown[0, rows, :] = (k_part + _bdot(crecv[rows, :], wrecv[0])
                             ).astype(BF16)
        kvown[1, rows, :] = (v_part + _bdot(crecv[rows, :], wrecv[1])
                             ).astype(BF16)

        kr_b = kr_scr[rows, :]
        qr_b = qr_scr[rows, :]
        for j in range(HG):
            cols = slice(j * Dh, (j + 1) * Dh)
            q_bh = q_scr[rows, cols]
            k_bh = kvown[0, rows, cols]
            qr_bh = qr_b[:, j * Dr:(j + 1) * Dr]
            s = (lax.dot_general(q_bh, k_bh, dims_t,
                                 preferred_element_type=F32)
                 + lax.dot_general(qr_bh, kr_b, dims_t,
                                   preferred_element_type=F32)) * SCALE
            e = jnp.exp(s)
            denom = jnp.sum(e, axis=-1, keepdims=True)
            o = _bdot(e.astype(BF16), kvown[1, rows, cols]) / denom
            obuf[0, rows, cols] = o.astype(BF16)
        _send(b)

    wo_fetch0.wait()
    out_ref[...] = _bdot(obuf[0], wobuf[0].astype(BF16))
    origin2 = (p + 2) % 4
    wo_fetch2 = pltpu.make_async_copy(
        wo_hbm.at[pl.ds(origin2 * CW, CW), :], wobuf.at[0], wo_sems.at[0])
    wo_fetch2.start()

    for h in range(1, 3):
        wo_fetchN = wo_fetch1 if h == 1 else wo_fetch2
        for half in range(B):
            ring[h - 1][half].wait()
            if half == 0:
                wo_fetchN.wait()
            rows = slice(half * S, (half + 1) * S)
            out_ref[rows, :] = out_ref[rows, :] + _bdot(
                obuf[h, rows, :], wobuf[h % 2].astype(BF16))
        if h == 1:
            origin3 = (p + 1) % 4
            wo_fetch3 = pltpu.make_async_copy(
                wo_hbm.at[pl.ds(origin3 * CW, CW), :], wobuf.at[1],
                wo_sems.at[1])
            wo_fetch3.start()

    for half in range(B):
        ring[2][half].wait()
        if half == 0:
            wo_fetch3.wait()
        rows = slice(half * S, (half + 1) * S)
        out_ref[rows, :] = out_ref[rows, :] + _bdot(
            obuf[3, rows, :], wobuf[1].astype(BF16))


def kernel(x, Wdkv, Wuk, Wuv, Wq, Wqr, Wkr, Wo):
    x2d = x.reshape(BS, D)

    out2d = pl.pallas_call(
        _body,
        out_shape=jax.ShapeDtypeStruct((BS, D), F32),
        in_specs=[pl.BlockSpec(memory_space=pltpu.VMEM)] * 4
        + [pl.BlockSpec(memory_space=pl.ANY)] * 2
        + [pl.BlockSpec(memory_space=pltpu.VMEM)]
        + [pl.BlockSpec(memory_space=pl.ANY)],
        out_specs=pl.BlockSpec(memory_space=pltpu.VMEM),
        scratch_shapes=[
            pltpu.VMEM((2, BS, CW), BF16),
            pltpu.VMEM((BS, DC), BF16),
            pltpu.VMEM((BS, DC), BF16),
            pltpu.VMEM((2, DC, CW), BF16),
            pltpu.VMEM((2, DC, CW), BF16),
            pltpu.VMEM((D, CW), F32),
            pltpu.VMEM((D, RW), F32),
            pltpu.VMEM((BS, CW), BF16),
            pltpu.VMEM((BS, RW), BF16),
            pltpu.VMEM((BS, Dr), BF16),
            pltpu.VMEM((4, BS, CW), BF16),
            pltpu.VMEM((2, CW, D), F32),
            pltpu.SemaphoreType.DMA((4,)),
            pltpu.SemaphoreType.DMA((2,)),
            pltpu.SemaphoreType.DMA((6,)),
            pltpu.SemaphoreType.DMA((6,)),
            pltpu.SemaphoreType.DMA((2,)),
            pltpu.SemaphoreType.DMA((2,)),
        ],
        compiler_params=pltpu.CompilerParams(
            collective_id=0, vmem_limit_bytes=60 * 1024 * 1024),
    )(x2d, Wdkv, Wuk, Wuv, Wq, Wqr, Wkr, Wo)
    return out2d.reshape(B, S, D)
